# baseline (speedup 1.0000x reference)
import jax
import jax.numpy as jnp
from jax.experimental import pallas as pl

N_SPOT = 50000
N_GENE = 50000
HID = 64


def _attention(x, att_W, att_q):
    xr = x.reshape(-1, 5, 512)
    outs = []
    for i in range(4):
        xw = jnp.einsum('nsd,dk->nsk', xr, att_W[i])
        a = jax.nn.leaky_relu(jnp.einsum('nsk,kl->nsl', xw, att_q[i]), 0.2)
        a = jax.nn.softmax(a, axis=1)
        outs.append(jnp.sum(xw * a, axis=1))
    return jnp.concatenate(outs, axis=1)


def _gat_conv(xs, xd, edge, p, n_dst):
    src, dst = edge[0], edge[1]
    hs = xs @ p['Ws']
    hd = xd @ p['Wd']
    e = jax.nn.leaky_relu((hs @ p['a_s'])[src] + (hd @ p['a_d'])[dst], 0.2)
    m = jax.ops.segment_max(e, dst, num_segments=n_dst)
    m = jnp.where(jnp.isfinite(m), m, 0.0)
    ex = jnp.exp(e - m[dst])
    den = jax.ops.segment_sum(ex, dst, num_segments=n_dst)
    alpha = ex / (den[dst] + 1e-16)
    out = jax.ops.segment_sum(hs[src] * alpha[:, None], dst, num_segments=n_dst)
    return out + p['b']


def _tail_kernel(xs_ref, xg_ref, ws_ref, bs_ref, wg_ref, bg_ref, os_ref, og_ref):
    os_ref[...] = xs_ref[...] @ ws_ref[...] + bs_ref[...]
    og_ref[...] = xg_ref[...] @ wg_ref[...] + bg_ref[...]


def kernel(x_spot, x_gene, edge_index_spot_gene, edge_index_gene_spot, params):
    xs = _attention(x_spot, params['att_W'], params['att_q'])
    xg = x_gene
    all_s, all_g = [], []
    for sg, gs in (('sg0', 'gs0'), ('sg1', 'gs1')):
        new_g = _gat_conv(xs, xg, edge_index_spot_gene, params[sg], N_GENE)
        new_s = _gat_conv(xg, xs, edge_index_gene_spot, params[gs], N_SPOT)
        xs, xg = new_s, new_g
        all_s.append(xs)
        all_g.append(xg)
        xs = jax.nn.relu(xs)
        xg = jax.nn.relu(xg)
    out_s, out_g = pl.pallas_call(
        _tail_kernel,
        out_shape=(jax.ShapeDtypeStruct((N_SPOT, 1), jnp.float32),
                   jax.ShapeDtypeStruct((N_GENE, 1), jnp.float32)),
        grid=(10,),
        in_specs=[pl.BlockSpec((N_SPOT // 10, HID), lambda i: (i, 0)),
                  pl.BlockSpec((N_GENE // 10, HID), lambda i: (i, 0)),
                  pl.BlockSpec((HID, 1), lambda i: (0, 0)),
                  pl.BlockSpec((1,), lambda i: (0,)),
                  pl.BlockSpec((HID, 1), lambda i: (0, 0)),
                  pl.BlockSpec((1,), lambda i: (0,))],
        out_specs=(pl.BlockSpec((N_SPOT // 10, 1), lambda i: (i, 0)),
                   pl.BlockSpec((N_GENE // 10, 1), lambda i: (i, 0))),
    )(xs, xg, params['lin_spot']['W'], params['lin_spot']['b'],
      params['lin_gene']['W'], params['lin_gene']['b'])
    xm_s = jnp.mean(jnp.stack(all_s, axis=1), axis=1)
    xm_g = jnp.mean(jnp.stack(all_g, axis=1), axis=1)
    return (xm_s, xm_g, out_s, out_g)


# speedup vs baseline: 1.8227x; 1.8227x over previous
import functools

import jax
import jax.numpy as jnp
from jax import lax
from jax.experimental import pallas as pl
from jax.experimental.pallas import tpu as pltpu
from jax.experimental.pallas import tpu_sc as plsc

N_SPOT = 50000
N_GENE = 50000
E = 800000
HID = 64

PH = 25088
P_ROWS = 25096
T_ROWS = 1568
AWID = 48
E_PAD = 802816
E_ROWS = 6272
CHUNKS_PER_TILE = 49


def _attn_body(x_ref, w_ref, q_ref, o_ref):
    xb = x_ref[...].astype(jnp.bfloat16)
    for i in range(4):
        wi = w_ref[i]
        xw = []
        logits = []
        for s in range(5):
            xw_s = jnp.dot(xb[:, 512 * s:512 * (s + 1)], wi,
                           preferred_element_type=jnp.float32)
            xw.append(xw_s)
            l_s = jnp.sum(xw_s * q_ref[i][None, :], axis=1, keepdims=True)
            logits.append(l_s)
        lg = jnp.concatenate(logits, axis=1)
        lg = jnp.where(lg > 0, lg, 0.2 * lg)
        lg = lg - jnp.max(lg, axis=1, keepdims=True)
        ex = jnp.exp(lg)
        a = ex / jnp.sum(ex, axis=1, keepdims=True)
        acc = a[:, 0:1] * xw[0]
        for s in range(1, 5):
            acc = acc + a[:, s:s + 1] * xw[s]
        o_ref[:, 128 * i:128 * (i + 1)] = acc


def _attention(x_spot, att_W, att_q):
    bn = 400
    grid = N_SPOT // bn
    return pl.pallas_call(
        _attn_body,
        out_shape=jax.ShapeDtypeStruct((N_SPOT, 512), jnp.float32),
        grid=(grid,),
        in_specs=[pl.BlockSpec((bn, 2560), lambda i: (i, 0)),
                  pl.BlockSpec((4, 512, 128), lambda i: (0, 0, 0)),
                  pl.BlockSpec((4, 128), lambda i: (0, 0))],
        out_specs=pl.BlockSpec((bn, 512), lambda i: (i, 0)),
    )(x_spot, att_W.astype(jnp.bfloat16), att_q[:, :, 0])


def _proj_body(xs_ref, xg_ref, ws_s, as_s, wd_s, ws_g, as_g, wd_g,
               h_s_ref, s_s_ref, d_g_ref, h_g_ref, s_g_ref, d_s_ref):
    xs = xs_ref[...]
    xg = xg_ref[...]

    zeros = jnp.zeros((xs.shape[0], 64), jnp.float32)

    h = jnp.dot(xs, ws_s[...], preferred_element_type=jnp.float32)
    h_s_ref[...] = jnp.concatenate([h, zeros], axis=1)
    s_s_ref[...] = jnp.sum(h * as_s[...][None, :], axis=1, keepdims=True)
    d_g_ref[...] = jnp.dot(xs, wd_g[...], preferred_element_type=jnp.float32)

    h = jnp.dot(xg, ws_g[...], preferred_element_type=jnp.float32)
    h_g_ref[...] = jnp.concatenate([h, zeros], axis=1)
    s_g_ref[...] = jnp.sum(h * as_g[...][None, :], axis=1, keepdims=True)
    d_s_ref[...] = jnp.dot(xg, wd_s[...], preferred_element_type=jnp.float32)


def _projections(xs, xg, p_sg, p_gs):
    bn = 1000
    grid = N_SPOT // bn
    din_s = xs.shape[1]
    din_g = xg.shape[1]
    wd_sg = (p_sg['Wd'] @ p_sg['a_d'])[:, None]
    wd_gs = (p_gs['Wd'] @ p_gs['a_d'])[:, None]
    return pl.pallas_call(
        _proj_body,
        out_shape=(jax.ShapeDtypeStruct((N_SPOT, 128), jnp.float32),
                   jax.ShapeDtypeStruct((N_SPOT, 1), jnp.float32),
                   jax.ShapeDtypeStruct((N_SPOT, 1), jnp.float32),
                   jax.ShapeDtypeStruct((N_GENE, 128), jnp.float32),
                   jax.ShapeDtypeStruct((N_GENE, 1), jnp.float32),
                   jax.ShapeDtypeStruct((N_GENE, 1), jnp.float32)),
        grid=(grid,),
        in_specs=[pl.BlockSpec((bn, din_s), lambda i: (i, 0)),
                  pl.BlockSpec((bn, din_g), lambda i: (i, 0)),
                  pl.BlockSpec((din_s, HID), lambda i: (0, 0)),
                  pl.BlockSpec((HID,), lambda i: (0,)),
                  pl.BlockSpec((din_g, 1), lambda i: (0, 0)),
                  pl.BlockSpec((din_g, HID), lambda i: (0, 0)),
                  pl.BlockSpec((HID,), lambda i: (0,)),
                  pl.BlockSpec((din_s, 1), lambda i: (0, 0))],
        out_specs=(pl.BlockSpec((bn, 128), lambda i: (i, 0)),
                   pl.BlockSpec((bn, 1), lambda i: (i, 0)),
                   pl.BlockSpec((bn, 1), lambda i: (i, 0)),
                   pl.BlockSpec((bn, 128), lambda i: (i, 0)),
                   pl.BlockSpec((bn, 1), lambda i: (i, 0)),
                   pl.BlockSpec((bn, 1), lambda i: (i, 0))),
    )(xs, xg, p_sg['Ws'], p_sg['a_s'], wd_sg,
      p_gs['Ws'], p_gs['a_s'], wd_gs)


def _div_bias(acc_ref, b_ref):
    num = jnp.concatenate([acc_ref[0, :, 0:32], acc_ref[1, :, 0:32]], axis=1)
    den = acc_ref[0, :, 32:33]
    return num / (den + 1e-16) + b_ref[...][None, :]


def _finalize0_body(accg_ref, accs_ref, bg_ref, bs_ref,
                    g_ref, s_ref, gr_ref, sr_ref):
    g = _div_bias(accg_ref, bg_ref)
    s = _div_bias(accs_ref, bs_ref)
    g_ref[...] = g
    s_ref[...] = s
    gr_ref[...] = jnp.maximum(g, 0.0)
    sr_ref[...] = jnp.maximum(s, 0.0)


def _finalize0(accg, accs, b_g, b_s):
    bn = 1000
    grid = N_GENE // bn
    return pl.pallas_call(
        _finalize0_body,
        out_shape=(jax.ShapeDtypeStruct((N_GENE, HID), jnp.float32),
                   jax.ShapeDtypeStruct((N_SPOT, HID), jnp.float32),
                   jax.ShapeDtypeStruct((N_GENE, HID), jnp.float32),
                   jax.ShapeDtypeStruct((N_SPOT, HID), jnp.float32)),
        grid=(grid,),
        in_specs=[pl.BlockSpec((2, bn, AWID), lambda i: (0, i, 0)),
                  pl.BlockSpec((2, bn, AWID), lambda i: (0, i, 0)),
                  pl.BlockSpec((HID,), lambda i: (0,)),
                  pl.BlockSpec((HID,), lambda i: (0,))],
        out_specs=(pl.BlockSpec((bn, HID), lambda i: (i, 0)),
                   pl.BlockSpec((bn, HID), lambda i: (i, 0)),
                   pl.BlockSpec((bn, HID), lambda i: (i, 0)),
                   pl.BlockSpec((bn, HID), lambda i: (i, 0))),
    )(accg, accs, b_g, b_s)


def _tail_body(accg_ref, accs_ref, bg_ref, bs_ref,
               s1_ref, g1_ref, lws_ref, lbs_ref, lwg_ref, lbg_ref,
               xms_ref, xmg_ref, os_ref, og_ref):
    g2 = _div_bias(accg_ref, bg_ref)
    s2 = _div_bias(accs_ref, bs_ref)
    xms_ref[...] = 0.5 * (s1_ref[...] + s2)
    xmg_ref[...] = 0.5 * (g1_ref[...] + g2)
    os_ref[...] = jnp.dot(jnp.maximum(s2, 0.0), lws_ref[...],
                          preferred_element_type=jnp.float32) + lbs_ref[...]
    og_ref[...] = jnp.dot(jnp.maximum(g2, 0.0), lwg_ref[...],
                          preferred_element_type=jnp.float32) + lbg_ref[...]


def _tail(accg, accs, b_g, b_s, s1, g1, lin_s, lin_g):
    bn = 1000
    grid = N_GENE // bn
    return pl.pallas_call(
        _tail_body,
        out_shape=(jax.ShapeDtypeStruct((N_SPOT, HID), jnp.float32),
                   jax.ShapeDtypeStruct((N_GENE, HID), jnp.float32),
                   jax.ShapeDtypeStruct((N_SPOT, 1), jnp.float32),
                   jax.ShapeDtypeStruct((N_GENE, 1), jnp.float32)),
        grid=(grid,),
        in_specs=[pl.BlockSpec((2, bn, AWID), lambda i: (0, i, 0)),
                  pl.BlockSpec((2, bn, AWID), lambda i: (0, i, 0)),
                  pl.BlockSpec((HID,), lambda i: (0,)),
                  pl.BlockSpec((HID,), lambda i: (0,)),
                  pl.BlockSpec((bn, HID), lambda i: (i, 0)),
                  pl.BlockSpec((bn, HID), lambda i: (i, 0)),
                  pl.BlockSpec((HID, 1), lambda i: (0, 0)),
                  pl.BlockSpec((1,), lambda i: (0,)),
                  pl.BlockSpec((HID, 1), lambda i: (0, 0)),
                  pl.BlockSpec((1,), lambda i: (0,))],
        out_specs=(pl.BlockSpec((bn, HID), lambda i: (i, 0)),
                   pl.BlockSpec((bn, HID), lambda i: (i, 0)),
                   pl.BlockSpec((bn, 1), lambda i: (i, 0)),
                   pl.BlockSpec((bn, 1), lambda i: (i, 0))),
    )(accg, accs, b_g, b_s, s1, g1,
      lin_s['W'], lin_s['b'], lin_g['W'], lin_g['b'])


def _finb_body(numg_ref, deng_ref, nums_ref, dens_ref, bg_ref, bs_ref,
               g_ref, s_ref, gr_ref, sr_ref):
    g = numg_ref[...] / (deng_ref[...] + 1e-16) + bg_ref[...][None, :]
    s = nums_ref[...] / (dens_ref[...] + 1e-16) + bs_ref[...][None, :]
    g_ref[...] = g
    s_ref[...] = s
    gr_ref[...] = jnp.maximum(g, 0.0)
    sr_ref[...] = jnp.maximum(s, 0.0)


def _finalize_b(numg, deng, nums, dens, b_g, b_s):
    bn = 1000
    grid = N_GENE // bn
    return pl.pallas_call(
        _finb_body,
        out_shape=(jax.ShapeDtypeStruct((N_GENE, HID), jnp.float32),
                   jax.ShapeDtypeStruct((N_SPOT, HID), jnp.float32),
                   jax.ShapeDtypeStruct((N_GENE, HID), jnp.float32),
                   jax.ShapeDtypeStruct((N_SPOT, HID), jnp.float32)),
        grid=(grid,),
        in_specs=[pl.BlockSpec((bn, HID), lambda i: (i, 0)),
                  pl.BlockSpec((bn, 1), lambda i: (i, 0)),
                  pl.BlockSpec((bn, HID), lambda i: (i, 0)),
                  pl.BlockSpec((bn, 1), lambda i: (i, 0)),
                  pl.BlockSpec((HID,), lambda i: (0,)),
                  pl.BlockSpec((HID,), lambda i: (0,))],
        out_specs=(pl.BlockSpec((bn, HID), lambda i: (i, 0)),
                   pl.BlockSpec((bn, HID), lambda i: (i, 0)),
                   pl.BlockSpec((bn, HID), lambda i: (i, 0)),
                   pl.BlockSpec((bn, HID), lambda i: (i, 0))),
    )(numg, deng, nums, dens, b_g, b_s)


def _tailb_body(numg_ref, deng_ref, nums_ref, dens_ref, bg_ref, bs_ref,
                s1_ref, g1_ref, lws_ref, lbs_ref, lwg_ref, lbg_ref,
                xms_ref, xmg_ref, os_ref, og_ref):
    g2 = numg_ref[...] / (deng_ref[...] + 1e-16) + bg_ref[...][None, :]
    s2 = nums_ref[...] / (dens_ref[...] + 1e-16) + bs_ref[...][None, :]
    xms_ref[...] = 0.5 * (s1_ref[...] + s2)
    xmg_ref[...] = 0.5 * (g1_ref[...] + g2)
    os_ref[...] = jnp.dot(jnp.maximum(s2, 0.0), lws_ref[...],
                          preferred_element_type=jnp.float32) + lbs_ref[...]
    og_ref[...] = jnp.dot(jnp.maximum(g2, 0.0), lwg_ref[...],
                          preferred_element_type=jnp.float32) + lbg_ref[...]


def _tail_b(numg, deng, nums, dens, b_g, b_s, s1, g1, lin_s, lin_g):
    bn = 1000
    grid = N_GENE // bn
    return pl.pallas_call(
        _tailb_body,
        out_shape=(jax.ShapeDtypeStruct((N_SPOT, HID), jnp.float32),
                   jax.ShapeDtypeStruct((N_GENE, HID), jnp.float32),
                   jax.ShapeDtypeStruct((N_SPOT, 1), jnp.float32),
                   jax.ShapeDtypeStruct((N_GENE, 1), jnp.float32)),
        grid=(grid,),
        in_specs=[pl.BlockSpec((bn, HID), lambda i: (i, 0)),
                  pl.BlockSpec((bn, 1), lambda i: (i, 0)),
                  pl.BlockSpec((bn, HID), lambda i: (i, 0)),
                  pl.BlockSpec((bn, 1), lambda i: (i, 0)),
                  pl.BlockSpec((HID,), lambda i: (0,)),
                  pl.BlockSpec((HID,), lambda i: (0,)),
                  pl.BlockSpec((bn, HID), lambda i: (i, 0)),
                  pl.BlockSpec((bn, HID), lambda i: (i, 0)),
                  pl.BlockSpec((HID, 1), lambda i: (0, 0)),
                  pl.BlockSpec((1,), lambda i: (0,)),
                  pl.BlockSpec((HID, 1), lambda i: (0, 0)),
                  pl.BlockSpec((1,), lambda i: (0,))],
        out_specs=(pl.BlockSpec((bn, HID), lambda i: (i, 0)),
                   pl.BlockSpec((bn, HID), lambda i: (i, 0)),
                   pl.BlockSpec((bn, 1), lambda i: (i, 0)),
                   pl.BlockSpec((bn, 1), lambda i: (i, 0))),
    )(numg, deng, nums, dens, b_g, b_s, s1, g1,
      lin_s['W'], lin_s['b'], lin_g['W'], lin_g['b'])


def _edge_numden(h128, svec, dvec, src, dst, n_dst):
    hs = h128[:, 0:HID]
    e = svec[src] + dvec[dst]
    e = jnp.maximum(e, 0.2 * e)
    w = jnp.exp(jnp.minimum(e, 80.0))
    den = jax.ops.segment_sum(w, dst, num_segments=n_dst)[:, None]
    num = jax.ops.segment_sum(hs[src] * w[:, None], dst, num_segments=n_dst)
    return num, den


def kernel(x_spot, x_gene, edge_index_spot_gene, edge_index_gene_spot, params):
    sg_src, sg_dst = edge_index_spot_gene[0], edge_index_spot_gene[1]
    gs_src, gs_dst = edge_index_gene_spot[0], edge_index_gene_spot[1]

    xs = _attention(x_spot, params['att_W'], params['att_q'])
    xg = x_gene

    h_sg, s_sg, d_gs, h_gs, s_gs, d_sg = _projections(
        xs, xg, params['sg0'], params['gs0'])
    numg, deng = _edge_numden(h_sg, s_sg[:, 0], d_sg[:, 0],
                              sg_src, sg_dst, N_GENE)
    nums, dens = _edge_numden(h_gs, s_gs[:, 0], d_gs[:, 0],
                              gs_src, gs_dst, N_SPOT)
    g1, s1, g1r, s1r = _finalize_b(numg, deng, nums, dens,
                                   params['sg0']['b'], params['gs0']['b'])

    h_sg, s_sg, d_gs, h_gs, s_gs, d_sg = _projections(
        s1r, g1r, params['sg1'], params['gs1'])
    numg, deng = _edge_numden(h_sg, s_sg[:, 0], d_sg[:, 0],
                              sg_src, sg_dst, N_GENE)
    nums, dens = _edge_numden(h_gs, s_gs[:, 0], d_gs[:, 0],
                              gs_src, gs_dst, N_SPOT)

    xm_s, xm_g, out_s, out_g = _tail_b(numg, deng, nums, dens,
                                       params['sg1']['b'], params['gs1']['b'],
                                       s1, g1,
                                       params['lin_spot'], params['lin_gene'])
    return (xm_s, xm_g, out_s, out_g)
